# baseline (speedup 1.0000x reference)
import functools
import math

import jax
import jax.numpy as jnp
import numpy as np
from jax import lax
from jax.experimental import pallas as pl
from jax.experimental.pallas import tpu as pltpu
from jax.experimental.pallas import tpu_sc as plsc

VOCAB = 100000
D_MODEL = 768
BATCH = 4
SEQ_LEN = 8192

_NC = 2
_NS = 16
_NW = _NC * _NS
_TOTAL = BATCH * SEQ_LEN
_PER_W = _TOTAL // _NW
_C = 32
_NCHUNK = _PER_W // _C
_LANES = D_MODEL // 16
_SCALE = math.sqrt(float(D_MODEL))


def _build_pe(seq_len, d_model):
    position = np.arange(seq_len, dtype=np.float32)[:, None]
    div_term = np.exp(
        np.arange(0, d_model, 2, dtype=np.float32) * (-np.log(10000.0) / d_model)
    )
    pe = np.zeros((seq_len, d_model), dtype=np.float32)
    pe[:, 0::2] = np.sin(position * div_term)
    pe[:, 1::2] = np.cos(position * div_term)
    return jnp.asarray(pe)


def _sc_body(ids_hbm, table_hbm, pe_hbm, out_hbm, idx_v, rows_v, pe_v, gsem):
    wid = lax.axis_index("s") * _NC + lax.axis_index("c")
    base = wid * _PER_W
    s0 = (wid % (SEQ_LEN // _PER_W)) * _PER_W

    pltpu.sync_copy(ids_hbm.at[pl.ds(base, _PER_W)], idx_v)

    def chunk(j, carry):
        off = j * _C
        pltpu.async_copy(
            table_hbm.at[idx_v.at[pl.ds(off, _C)]], rows_v, gsem
        ).wait()
        pltpu.sync_copy(pe_hbm.at[pl.ds(s0 + off, _C)], pe_v)

        def row(r, c2):
            for g in range(_LANES):
                sl = pl.ds(g * 16, 16)
                rows_v[r, sl] = rows_v[r, sl] * _SCALE + pe_v[r, sl]
            return c2

        lax.fori_loop(0, _C, row, 0)
        pltpu.sync_copy(rows_v, out_hbm.at[pl.ds(base + off, _C)])
        return carry

    lax.fori_loop(0, _NCHUNK, chunk, 0)


@jax.jit
def _combined_embedding(ids_flat, token_table, pe):
    mesh = plsc.VectorSubcoreMesh(core_axis_name="c", subcore_axis_name="s")
    fn = pl.kernel(
        _sc_body,
        out_type=jax.ShapeDtypeStruct((_TOTAL, D_MODEL), jnp.float32),
        mesh=mesh,
        scratch_types=[
            pltpu.VMEM((_PER_W,), jnp.int32),
            pltpu.VMEM((_C, D_MODEL), jnp.float32),
            pltpu.VMEM((_C, D_MODEL), jnp.float32),
            pltpu.SemaphoreType.DMA,
        ],
    )
    return fn(ids_flat, token_table, pe)


def kernel(input_ids, token_table):
    ids_flat = input_ids.reshape(-1).astype(jnp.int32)
    pe = _build_pe(SEQ_LEN, D_MODEL)
    out = _combined_embedding(ids_flat, token_table, pe)
    return out.reshape(BATCH, SEQ_LEN, D_MODEL)


# speedup vs baseline: 1.4774x; 1.4774x over previous
import math

import jax
import jax.numpy as jnp
import numpy as np
from jax import lax
from jax.experimental import pallas as pl
from jax.experimental.pallas import tpu as pltpu
from jax.experimental.pallas import tpu_sc as plsc

VOCAB = 100000
D_MODEL = 768
BATCH = 4
SEQ_LEN = 8192

_NC = 2
_NS = 16
_NW = _NC * _NS
_PPW = SEQ_LEN // _NW
_C = 16
_NJ = _PPW // _C
_LANES = D_MODEL // 16
_SCALE = math.sqrt(float(D_MODEL))


def _build_pe(seq_len, d_model):
    position = np.arange(seq_len, dtype=np.float32)[:, None]
    div_term = np.exp(
        np.arange(0, d_model, 2, dtype=np.float32) * (-np.log(10000.0) / d_model)
    )
    pe = np.zeros((seq_len, d_model), dtype=np.float32)
    pe[:, 0::2] = np.sin(position * div_term)
    pe[:, 1::2] = np.cos(position * div_term)
    return jnp.asarray(pe)


def _sc_body(ids_hbm, table_hbm, pe_hbm, out_hbm, idx_v,
             r0, r1, r2, r3, p0, p1,
             sg0, sg1, sg2, sg3, so0, so1, so2, so3, sp0, sp1):
    rbuf = [r0, r1, r2, r3]
    pbuf = [p0, p1]
    sg = [sg0, sg1, sg2, sg3]
    so = [so0, so1, so2, so3]
    sp = [sp0, sp1]

    wid = lax.axis_index("s") * _NC + lax.axis_index("c")
    pos0 = wid * _PPW

    for b in range(BATCH):
        pltpu.sync_copy(
            ids_hbm.at[pl.ds(b * SEQ_LEN + pos0, _PPW)],
            idx_v.at[pl.ds(b * _PPW, _PPW)],
        )

    def start_gather(jn, bn):
        pltpu.async_copy(
            table_hbm.at[idx_v.at[pl.ds(bn * _PPW + jn * _C, _C)]],
            rbuf[bn], sg[bn],
        )

    def wait_gather(jn, bn):
        pltpu.make_async_copy(
            table_hbm.at[idx_v.at[pl.ds(bn * _PPW + jn * _C, _C)]],
            rbuf[bn], sg[bn],
        ).wait()

    def start_pe(jn, ps):
        pltpu.async_copy(pe_hbm.at[pl.ds(pos0 + jn * _C, _C)], pbuf[ps], sp[ps])

    def wait_out(bn):
        pltpu.make_async_copy(rbuf[bn], out_hbm.at[pl.ds(0, _C)], so[bn]).wait()

    start_pe(0, 0)
    start_gather(0, 0)

    @pl.loop(0, _NJ, step=2)
    def _(jj):
        for dj in range(2):
            j = jj + dj
            for b in range(BATCH):
                bn = (b + 1) % BATCH
                if b == BATCH - 1:
                    wait_out(bn)

                    @pl.when(j + 1 < _NJ)
                    def _():
                        start_gather(j + 1, bn)
                else:
                    @pl.when(j > 0)
                    def _():
                        wait_out(bn)

                    start_gather(j, bn)

                wait_gather(j, b)
                if b == 0:
                    pltpu.make_async_copy(
                        pe_hbm.at[pl.ds(0, _C)], pbuf[dj], sp[dj]
                    ).wait()

                    @pl.when(j + 1 < _NJ)
                    def _():
                        start_pe(j + 1, (dj + 1) % 2)

                @pl.loop(0, _C)
                def _(r):
                    for g in range(_LANES):
                        sl = pl.ds(g * 16, 16)
                        rbuf[b][r, sl] = rbuf[b][r, sl] * _SCALE + pbuf[dj][r, sl]

                pltpu.async_copy(
                    rbuf[b],
                    out_hbm.at[pl.ds(b * SEQ_LEN + pos0 + j * _C, _C)],
                    so[b],
                )

    for b in range(1, BATCH):
        wait_out(b)


@jax.jit
def _combined_embedding(ids_flat, token_table, pe):
    mesh = plsc.VectorSubcoreMesh(core_axis_name="c", subcore_axis_name="s")
    fn = pl.kernel(
        _sc_body,
        out_type=jax.ShapeDtypeStruct((BATCH * SEQ_LEN, D_MODEL), jnp.float32),
        mesh=mesh,
        scratch_types=[
            pltpu.VMEM((BATCH * _PPW,), jnp.int32),
            pltpu.VMEM((_C, D_MODEL), jnp.float32),
            pltpu.VMEM((_C, D_MODEL), jnp.float32),
            pltpu.VMEM((_C, D_MODEL), jnp.float32),
            pltpu.VMEM((_C, D_MODEL), jnp.float32),
            pltpu.VMEM((_C, D_MODEL), jnp.float32),
            pltpu.VMEM((_C, D_MODEL), jnp.float32),
        ] + [pltpu.SemaphoreType.DMA] * 10,
    )
    return fn(ids_flat, token_table, pe)


def kernel(input_ids, token_table):
    ids_flat = input_ids.reshape(-1).astype(jnp.int32)
    pe = _build_pe(SEQ_LEN, D_MODEL)
    out = _combined_embedding(ids_flat, token_table, pe)
    return out.reshape(BATCH, SEQ_LEN, D_MODEL)


# speedup vs baseline: 1.6792x; 1.1366x over previous
import math

import jax
import jax.numpy as jnp
import numpy as np
from jax import lax
from jax.experimental import pallas as pl
from jax.experimental.pallas import tpu as pltpu
from jax.experimental.pallas import tpu_sc as plsc

VOCAB = 100000
D_MODEL = 768
BATCH = 4
SEQ_LEN = 8192

_NC = 2
_NS = 16
_NW = _NC * _NS
_PPW = SEQ_LEN // _NW
_C = 16
_NJ = _PPW // _C
_LANES = D_MODEL // 16
_SCALE = math.sqrt(float(D_MODEL))


def _build_pe(seq_len, d_model):
    position = np.arange(seq_len, dtype=np.float32)[:, None]
    div_term = np.exp(
        np.arange(0, d_model, 2, dtype=np.float32) * (-np.log(10000.0) / d_model)
    )
    pe = np.zeros((seq_len, d_model), dtype=np.float32)
    pe[:, 0::2] = np.sin(position * div_term)
    pe[:, 1::2] = np.cos(position * div_term)
    return jnp.asarray(pe)


def _sc_body(ids_hbm, table_hbm, pe_hbm, out_hbm, idx_v,
             r00, r01, r02, r03, r10, r11, r12, r13, p0, p1,
             sg00, sg01, sg02, sg03, sg10, sg11, sg12, sg13,
             so00, so01, so02, so03, so10, so11, so12, so13,
             sp0, sp1):
    rbuf = [[r00, r01, r02, r03], [r10, r11, r12, r13]]
    pbuf = [p0, p1]
    sg = [[sg00, sg01, sg02, sg03], [sg10, sg11, sg12, sg13]]
    so = [[so00, so01, so02, so03], [so10, so11, so12, so13]]
    sp = [sp0, sp1]

    wid = lax.axis_index("s") * _NC + lax.axis_index("c")
    pos0 = wid * _PPW

    for b in range(BATCH):
        pltpu.sync_copy(
            ids_hbm.at[b, pl.ds(pos0, _PPW)],
            idx_v.at[pl.ds(b * _PPW, _PPW)],
        )

    def start_chunk(j, ring):
        pltpu.async_copy(pe_hbm.at[pl.ds(pos0 + j * _C, _C)], pbuf[ring], sp[ring])
        for b in range(BATCH):
            pltpu.async_copy(
                table_hbm.at[idx_v.at[pl.ds(b * _PPW + j * _C, _C)]],
                rbuf[ring][b], sg[ring][b],
            )

    def wait_outs(ring):
        for b in range(BATCH):
            pltpu.make_async_copy(
                rbuf[ring][b], out_hbm.at[b, pl.ds(0, _C), :], so[ring][b]
            ).wait()

    start_chunk(0, 0)

    @pl.loop(0, _NJ, step=2)
    def _(jj):
        for ring in range(2):
            j = jj + ring
            nring = (ring + 1) % 2

            @pl.when(j + 1 < _NJ)
            def _():
                @pl.when(j >= 1)
                def _():
                    wait_outs(nring)

                start_chunk(j + 1, nring)

            pltpu.make_async_copy(
                pe_hbm.at[pl.ds(0, _C)], pbuf[ring], sp[ring]
            ).wait()
            for b in range(BATCH):
                pltpu.make_async_copy(
                    table_hbm.at[idx_v.at[pl.ds(b * _PPW + j * _C, _C)]],
                    rbuf[ring][b], sg[ring][b],
                ).wait()

            @pl.loop(0, _C)
            def _(r):
                for g in range(_LANES):
                    sl = pl.ds(g * 16, 16)
                    pv = pbuf[ring][r, sl]
                    for b in range(BATCH):
                        rbuf[ring][b][r, sl] = rbuf[ring][b][r, sl] * _SCALE + pv

            for b in range(BATCH):
                pltpu.async_copy(
                    rbuf[ring][b],
                    out_hbm.at[b, pl.ds(pos0 + j * _C, _C), :],
                    so[ring][b],
                )

    wait_outs(0)
    wait_outs(1)


@jax.jit
def _combined_embedding(input_ids, token_table, pe):
    mesh = plsc.VectorSubcoreMesh(core_axis_name="c", subcore_axis_name="s")
    fn = pl.kernel(
        _sc_body,
        out_type=jax.ShapeDtypeStruct((BATCH, SEQ_LEN, D_MODEL), jnp.float32),
        mesh=mesh,
        scratch_types=[
            pltpu.VMEM((BATCH * _PPW,), jnp.int32),
        ] + [pltpu.VMEM((_C, D_MODEL), jnp.float32)] * 10
          + [pltpu.SemaphoreType.DMA] * 18,
    )
    return fn(input_ids, token_table, pe)


def kernel(input_ids, token_table):
    pe = _build_pe(SEQ_LEN, D_MODEL)
    return _combined_embedding(input_ids.astype(jnp.int32), token_table, pe)


# speedup vs baseline: 2.0136x; 1.1992x over previous
import math

import jax
import jax.numpy as jnp
import numpy as np
from jax import lax
from jax.experimental import pallas as pl
from jax.experimental.pallas import tpu as pltpu
from jax.experimental.pallas import tpu_sc as plsc

VOCAB = 100000
D_MODEL = 768
BATCH = 4
SEQ_LEN = 8192

_NC = 2
_NS = 16
_NW = _NC * _NS
_PPW = SEQ_LEN // _NW
_C = 16
_NJ = _PPW // _C
_LANES = D_MODEL // 16
_SCALE = math.sqrt(float(D_MODEL))


def _build_pe(seq_len, d_model):
    position = np.arange(seq_len, dtype=np.float32)[:, None]
    div_term = np.exp(
        np.arange(0, d_model, 2, dtype=np.float32) * (-np.log(10000.0) / d_model)
    )
    pe = np.zeros((seq_len, d_model), dtype=np.float32)
    pe[:, 0::2] = np.sin(position * div_term)
    pe[:, 1::2] = np.cos(position * div_term)
    return jnp.asarray(pe)


_PE_CACHE = []


def _pe_device():
    if not _PE_CACHE:
        _PE_CACHE.append(_build_pe(SEQ_LEN, D_MODEL))
    return _PE_CACHE[0]


def _sc_body(ids_hbm, table_hbm, pe_hbm, out_hbm, idx_v,
             r00, r01, r02, r03, r10, r11, r12, r13, p0, p1,
             sg00, sg01, sg02, sg03, sg10, sg11, sg12, sg13,
             so00, so01, so02, so03, so10, so11, so12, so13,
             sp0, sp1):
    rbuf = [[r00, r01, r02, r03], [r10, r11, r12, r13]]
    pbuf = [p0, p1]
    sg = [[sg00, sg01, sg02, sg03], [sg10, sg11, sg12, sg13]]
    so = [[so00, so01, so02, so03], [so10, so11, so12, so13]]
    sp = [sp0, sp1]

    wid = lax.axis_index("s") * _NC + lax.axis_index("c")
    pos0 = wid * _PPW

    for b in range(BATCH):
        pltpu.async_copy(
            ids_hbm.at[b, pl.ds(pos0, _PPW)],
            idx_v.at[pl.ds(b * _PPW, _PPW)],
            sp0,
        )
    for b in range(BATCH):
        pltpu.make_async_copy(
            ids_hbm.at[0, pl.ds(0, _PPW)],
            idx_v.at[pl.ds(b * _PPW, _PPW)],
            sp0,
        ).wait()

    def start_chunk(j, ring):
        pltpu.async_copy(pe_hbm.at[pl.ds(pos0 + j * _C, _C)], pbuf[ring], sp[ring])
        for b in range(BATCH):
            pltpu.async_copy(
                table_hbm.at[idx_v.at[pl.ds(b * _PPW + j * _C, _C)]],
                rbuf[ring][b], sg[ring][b],
            )

    def wait_outs(ring):
        for b in range(BATCH):
            pltpu.make_async_copy(
                rbuf[ring][b], out_hbm.at[b, pl.ds(0, _C), :], so[ring][b]
            ).wait()

    start_chunk(0, 0)

    @pl.loop(0, _NJ, step=2)
    def _(jj):
        for ring in range(2):
            j = jj + ring
            nring = (ring + 1) % 2

            @pl.when(j + 1 < _NJ)
            def _():
                @pl.when(j >= 1)
                def _():
                    wait_outs(nring)

                start_chunk(j + 1, nring)

            pltpu.make_async_copy(
                pe_hbm.at[pl.ds(0, _C)], pbuf[ring], sp[ring]
            ).wait()
            for b in range(BATCH):
                pltpu.make_async_copy(
                    table_hbm.at[idx_v.at[pl.ds(b * _PPW + j * _C, _C)]],
                    rbuf[ring][b], sg[ring][b],
                ).wait()

            @pl.loop(0, _C, unroll=2)
            def _(r):
                for g in range(_LANES):
                    sl = pl.ds(g * 16, 16)
                    pv = pbuf[ring][r, sl]
                    for b in range(BATCH):
                        rbuf[ring][b][r, sl] = rbuf[ring][b][r, sl] * _SCALE + pv

            for b in range(BATCH):
                pltpu.async_copy(
                    rbuf[ring][b],
                    out_hbm.at[b, pl.ds(pos0 + j * _C, _C), :],
                    so[ring][b],
                )

    wait_outs(0)
    wait_outs(1)


@jax.jit
def _combined_embedding(input_ids, token_table, pe):
    mesh = plsc.VectorSubcoreMesh(core_axis_name="c", subcore_axis_name="s")
    fn = pl.kernel(
        _sc_body,
        out_type=jax.ShapeDtypeStruct((BATCH, SEQ_LEN, D_MODEL), jnp.float32),
        mesh=mesh,
        scratch_types=[
            pltpu.VMEM((BATCH * _PPW,), jnp.int32),
        ] + [pltpu.VMEM((_C, D_MODEL), jnp.float32)] * 10
          + [pltpu.SemaphoreType.DMA] * 18,
    )
    return fn(input_ids, token_table, pe)


def kernel(input_ids, token_table):
    return _combined_embedding(
        input_ids.astype(jnp.int32), token_table, _pe_device()
    )
